# Initial kernel scaffold; baseline (speedup 1.0000x reference)
import jax
import jax.numpy as jnp
from jax.experimental import pallas as pl


def kernel(inputs, user_table, movie_table):
    raise NotImplementedError("write your pallas kernel here")


# speedup vs baseline: 1.3940x; 1.3940x over previous
import functools

import jax
import jax.numpy as jnp
from jax import lax
from jax.experimental import pallas as pl
from jax.experimental.pallas import tpu as pltpu
from jax.experimental.pallas import tpu_sc as plsc

B = 16384
D = 128
NUM_WORKERS = 32
ROWS_PER_WORKER = B // NUM_WORKERS
CHUNK = 128
NUM_CHUNKS = ROWS_PER_WORKER // CHUNK
LANES = 16
D_VECS = D // LANES


def _sc_kernel(uidx_hbm, midx_hbm, utab_hbm, mtab_hbm, out_hbm,
               uidx_v, midx_v, urows, mrows, outv, sem_u, sem_m):
    wid = lax.axis_index("s") * 2 + lax.axis_index("c")
    pltpu.sync_copy(uidx_hbm.at[wid], uidx_v)
    pltpu.sync_copy(midx_hbm.at[wid], midx_v)

    iota = lax.iota(jnp.int32, LANES)

    for c in range(NUM_CHUNKS):
        cu = pltpu.async_copy(utab_hbm.at[uidx_v.at[c]], urows, sem_u)
        cm = pltpu.async_copy(mtab_hbm.at[midx_v.at[c]], mrows, sem_m)
        cu.wait()
        cm.wait()

        def group_body(g, _):
            row0 = g * LANES

            def row_body(r, accv):
                row = row0 + r
                p = urows[row, pl.ds(0, LANES)] * mrows[row, pl.ds(0, LANES)]
                for j in range(1, D_VECS):
                    p = p + (urows[row, pl.ds(j * LANES, LANES)]
                             * mrows[row, pl.ds(j * LANES, LANES)])
                return jnp.where(iota == r, jnp.sum(p), accv)

            accv = lax.fori_loop(0, LANES, row_body,
                                 jnp.zeros((LANES,), jnp.float32))
            outv[pl.ds(c * CHUNK + row0, LANES)] = accv
            return 0

        lax.fori_loop(0, CHUNK // LANES, group_body, 0)

    base = wid * ROWS_PER_WORKER
    pltpu.sync_copy(outv, out_hbm.at[pl.ds(base, ROWS_PER_WORKER)])


@jax.jit
def _run(uidx, midx, user_table, movie_table):
    mesh = plsc.VectorSubcoreMesh(core_axis_name="c", subcore_axis_name="s")
    fn = functools.partial(
        pl.kernel,
        mesh=mesh,
        compiler_params=pltpu.CompilerParams(needs_layout_passes=False),
        out_type=jax.ShapeDtypeStruct((B,), jnp.float32),
        scratch_types=[
            pltpu.VMEM((NUM_CHUNKS, CHUNK), jnp.int32),
            pltpu.VMEM((NUM_CHUNKS, CHUNK), jnp.int32),
            pltpu.VMEM((CHUNK, D), jnp.float32),
            pltpu.VMEM((CHUNK, D), jnp.float32),
            pltpu.VMEM((ROWS_PER_WORKER,), jnp.float32),
            pltpu.SemaphoreType.DMA,
            pltpu.SemaphoreType.DMA,
        ],
    )(_sc_kernel)
    return fn(uidx, midx, user_table, movie_table)


def kernel(inputs, user_table, movie_table):
    idx = inputs.astype(jnp.int32)
    uidx = idx[:, 0].reshape(NUM_WORKERS, NUM_CHUNKS, CHUNK)
    midx = idx[:, 1].reshape(NUM_WORKERS, NUM_CHUNKS, CHUNK)
    out = _run(uidx, midx, user_table, movie_table)
    return out.reshape(B, 1)
